# baseline (speedup 1.0000x reference)
import jax
import jax.numpy as jnp
from jax import lax
from jax.experimental import pallas as pl
from jax.experimental.pallas import tpu as pltpu
from jax.experimental.pallas import tpu_sc as plsc

B = 8192
W = 4096
C = 4096
L = 16

NC = 2
NS = 16
NW = NC * NS

G = 4
NBUF = 2
ROWS_PER_W = B // NW
STEPS = ROWS_PER_W // G
CCHUNKS = C // L


def _body(inp_flat, wv_hbm, idx_hbm, out_hbm, idx_v, wv_v,
          in_v0, in_v1, out_v0, out_v1,
          sem_in0, sem_in1, sem_cp, sem_go0, sem_go1):
    in_bufs = (in_v0, in_v1)
    out_bufs = (out_v0, out_v1)
    sem_ins = (sem_in0, sem_in1)
    sem_gos = (sem_go0, sem_go1)

    cid = lax.axis_index("c")
    sid = lax.axis_index("s")
    wid = sid * NC + cid
    base = wid * ROWS_PER_W

    pltpu.sync_copy(idx_hbm, idx_v)
    pltpu.sync_copy(wv_hbm, wv_v)

    def in_src(t):
        return inp_flat.at[pl.ds((base + t * G) * W, G * W)]

    for b in range(NBUF):
        pltpu.async_copy(in_src(b), in_bufs[b], sem_ins[b])

    @pl.loop(0, STEPS, step=NBUF)
    def _step(t0):
        for b in range(NBUF):
            t = t0 + b
            row0 = base + t * G
            inb = in_bufs[b]
            outb = out_bufs[b]

            pltpu.make_async_copy(in_src(t), inb, sem_ins[b]).wait()

            cps = [
                pltpu.async_copy(
                    inb.at[pl.ds(g * W, W)], out_hbm.at[row0 + g, 0], sem_cp
                )
                for g in range(G)
            ]

            @pl.when(t >= NBUF)
            def _():
                for g in range(G):
                    pltpu.make_async_copy(
                        outb.at[pl.ds(g * C, C)], out_hbm.at[row0 + g, 1], sem_gos[b]
                    ).wait()

            @pl.loop(0, CCHUNKS)
            def _chunk(j):
                sl = pl.ds(j * L, L)
                ids = idx_v[sl]
                w = wv_v[sl]
                for g in range(G):
                    vals = plsc.load_gather(inb.at[pl.ds(g * W, W)], [ids])
                    outb[pl.ds(g * C + j * L, L)] = vals * w

            for g in range(G):
                pltpu.async_copy(
                    outb.at[pl.ds(g * C, C)], out_hbm.at[row0 + g, 1], sem_gos[b]
                )

            for cp in cps:
                cp.wait()

            @pl.when(t + NBUF < STEPS)
            def _():
                pltpu.async_copy(in_src(t + NBUF), inb, sem_ins[b])

    for b in range(NBUF):
        for g in range(G):
            pltpu.make_async_copy(
                out_bufs[b].at[pl.ds(g * C, C)], out_hbm.at[base + g, 1], sem_gos[b]
            ).wait()


@jax.jit
def kernel(input, weight_values, parent_idx):
    mesh = plsc.VectorSubcoreMesh(core_axis_name="c", subcore_axis_name="s")
    run = pl.kernel(
        _body,
        out_type=jax.ShapeDtypeStruct((B, 2, W), jnp.float32),
        mesh=mesh,
        scratch_types=[
            pltpu.VMEM((C,), jnp.int32),
            pltpu.VMEM((C,), jnp.float32),
            pltpu.VMEM((G * W,), jnp.float32),
            pltpu.VMEM((G * W,), jnp.float32),
            pltpu.VMEM((G * C,), jnp.float32),
            pltpu.VMEM((G * C,), jnp.float32),
            pltpu.SemaphoreType.DMA,
            pltpu.SemaphoreType.DMA,
            pltpu.SemaphoreType.DMA,
            pltpu.SemaphoreType.DMA,
            pltpu.SemaphoreType.DMA,
        ],
        compiler_params=pltpu.CompilerParams(needs_layout_passes=False),
    )
    out = run(input.reshape(-1), weight_values, parent_idx.astype(jnp.int32))
    return out.reshape(B, W + C)


# speedup vs baseline: 1.6162x; 1.6162x over previous
import jax
import jax.numpy as jnp
from jax import lax
from jax.experimental import pallas as pl
from jax.experimental.pallas import tpu as pltpu
from jax.experimental.pallas import tpu_sc as plsc

B = 8192
W = 4096
C = 4096
L = 16

NC = 2
NS = 16
NW = NC * NS

G = 4
NBUF = 2
ROWS_PER_W = B // NW
STEPS = ROWS_PER_W // G
CCHUNKS = C // L


def _body(inp_hbm, wv_hbm, idx_hbm, out_hbm, idx_v, wv_v,
          in_v0, in_v1, out_v0, out_v1,
          sem_in0, sem_in1, sem_cp, sem_go0, sem_go1):
    in_bufs = (in_v0, in_v1)
    out_bufs = (out_v0, out_v1)
    sem_ins = (sem_in0, sem_in1)
    sem_gos = (sem_go0, sem_go1)

    cid = lax.axis_index("c")
    sid = lax.axis_index("s")
    wid = sid * NC + cid
    base = wid * ROWS_PER_W

    pltpu.sync_copy(idx_hbm, idx_v)
    pltpu.sync_copy(wv_hbm, wv_v)

    def fire_in(t, b):
        row0 = base + t * G
        for g in range(G):
            pltpu.async_copy(
                inp_hbm.at[row0 + g], in_bufs[b].at[pl.ds(g * W, W)], sem_ins[b]
            )

    def wait_in(t, b):
        row0 = base + t * G
        for g in range(G):
            pltpu.make_async_copy(
                inp_hbm.at[row0 + g], in_bufs[b].at[pl.ds(g * W, W)], sem_ins[b]
            ).wait()

    for b in range(NBUF):
        fire_in(b, b)

    @pl.loop(0, STEPS, step=NBUF)
    def _step(t0):
        for b in range(NBUF):
            t = t0 + b
            row0 = base + t * G
            inb = in_bufs[b]
            outb = out_bufs[b]

            wait_in(t, b)

            cps = [
                pltpu.async_copy(
                    inb.at[pl.ds(g * W, W)], out_hbm.at[row0 + g, pl.ds(0, W)], sem_cp
                )
                for g in range(G)
            ]

            @pl.when(t >= NBUF)
            def _():
                for g in range(G):
                    pltpu.make_async_copy(
                        outb.at[pl.ds(g * C, C)],
                        out_hbm.at[row0 + g, pl.ds(W, C)],
                        sem_gos[b],
                    ).wait()

            @pl.loop(0, CCHUNKS)
            def _chunk(j):
                sl = pl.ds(j * L, L)
                ids = idx_v[sl]
                w = wv_v[sl]
                for g in range(G):
                    vals = plsc.load_gather(inb.at[pl.ds(g * W, W)], [ids])
                    outb[pl.ds(g * C + j * L, L)] = vals * w

            for g in range(G):
                pltpu.async_copy(
                    outb.at[pl.ds(g * C, C)],
                    out_hbm.at[row0 + g, pl.ds(W, C)],
                    sem_gos[b],
                )

            for cp in cps:
                cp.wait()

            @pl.when(t + NBUF < STEPS)
            def _():
                fire_in(t + NBUF, b)

    for b in range(NBUF):
        for g in range(G):
            pltpu.make_async_copy(
                out_bufs[b].at[pl.ds(g * C, C)],
                out_hbm.at[base + g, pl.ds(W, C)],
                sem_gos[b],
            ).wait()


@jax.jit
def kernel(input, weight_values, parent_idx):
    mesh = plsc.VectorSubcoreMesh(core_axis_name="c", subcore_axis_name="s")
    run = pl.kernel(
        _body,
        out_type=jax.ShapeDtypeStruct((B, W + C), jnp.float32),
        mesh=mesh,
        scratch_types=[
            pltpu.VMEM((C,), jnp.int32),
            pltpu.VMEM((C,), jnp.float32),
            pltpu.VMEM((G * W,), jnp.float32),
            pltpu.VMEM((G * W,), jnp.float32),
            pltpu.VMEM((G * C,), jnp.float32),
            pltpu.VMEM((G * C,), jnp.float32),
            pltpu.SemaphoreType.DMA,
            pltpu.SemaphoreType.DMA,
            pltpu.SemaphoreType.DMA,
            pltpu.SemaphoreType.DMA,
            pltpu.SemaphoreType.DMA,
        ],
        compiler_params=pltpu.CompilerParams(needs_layout_passes=False),
    )
    return run(input, weight_values, parent_idx.astype(jnp.int32))


# speedup vs baseline: 4.3682x; 2.7027x over previous
import jax
import jax.numpy as jnp
from jax import lax
from jax.experimental import pallas as pl
from jax.experimental.pallas import tpu as pltpu
from jax.experimental.pallas import tpu_sc as plsc

B = 8192
W = 4096
C = 4096
L = 16

NC = 2
NS = 16
NW = NC * NS

G = 4
NBUF = 2
ROWS_PER_W = B // NW
STEPS = ROWS_PER_W // G
CCHUNKS = C // L


def _body(inp_hbm, wv_hbm, idx_hbm, out_hbm, idx_v, wv_v,
          in_v0, in_v1, out_v0, out_v1,
          sem_in0, sem_in1, sem_cp, sem_go0, sem_go1):
    in_bufs = (in_v0, in_v1)
    out_bufs = (out_v0, out_v1)
    sem_ins = (sem_in0, sem_in1)
    sem_gos = (sem_go0, sem_go1)

    cid = lax.axis_index("c")
    sid = lax.axis_index("s")
    wid = sid * NC + cid
    base = wid * ROWS_PER_W

    pltpu.sync_copy(idx_hbm, idx_v)
    pltpu.sync_copy(wv_hbm, wv_v)

    def fire_in(t, b):
        row0 = base + t * G
        for g in range(G):
            pltpu.async_copy(
                inp_hbm.at[row0 + g], in_bufs[b].at[pl.ds(g * W, W)], sem_ins[b]
            )

    def wait_in(t, b):
        row0 = base + t * G
        for g in range(G):
            pltpu.make_async_copy(
                inp_hbm.at[row0 + g], in_bufs[b].at[pl.ds(g * W, W)], sem_ins[b]
            ).wait()

    for b in range(NBUF):
        fire_in(b, b)

    @pl.loop(0, STEPS, step=NBUF)
    def _step(t0):
        for b in range(NBUF):
            t = t0 + b
            row0 = base + t * G
            inb = in_bufs[b]
            outb = out_bufs[b]

            wait_in(t, b)

            cps = [
                pltpu.async_copy(
                    inb.at[pl.ds(g * W, W)], out_hbm.at[row0 + g, pl.ds(0, W)], sem_cp
                )
                for g in range(G)
            ]

            @pl.when(t >= NBUF)
            def _():
                for g in range(G):
                    pltpu.make_async_copy(
                        outb.at[pl.ds(g * C, C)],
                        out_hbm.at[row0 + g, pl.ds(W, C)],
                        sem_gos[b],
                    ).wait()

            @plsc.parallel_loop(0, CCHUNKS, unroll=4)
            def _chunk(j):
                sl = pl.ds(j * L, L)
                ids = idx_v[sl]
                w = wv_v[sl]
                for g in range(G):
                    vals = plsc.load_gather(inb.at[pl.ds(g * W, W)], [ids])
                    outb[pl.ds(g * C + j * L, L)] = vals * w

            for g in range(G):
                pltpu.async_copy(
                    outb.at[pl.ds(g * C, C)],
                    out_hbm.at[row0 + g, pl.ds(W, C)],
                    sem_gos[b],
                )

            for cp in cps:
                cp.wait()

            @pl.when(t + NBUF < STEPS)
            def _():
                fire_in(t + NBUF, b)

    for b in range(NBUF):
        for g in range(G):
            pltpu.make_async_copy(
                out_bufs[b].at[pl.ds(g * C, C)],
                out_hbm.at[base + g, pl.ds(W, C)],
                sem_gos[b],
            ).wait()


@jax.jit
def kernel(input, weight_values, parent_idx):
    mesh = plsc.VectorSubcoreMesh(core_axis_name="c", subcore_axis_name="s")
    run = pl.kernel(
        _body,
        out_type=jax.ShapeDtypeStruct((B, W + C), jnp.float32),
        mesh=mesh,
        scratch_types=[
            pltpu.VMEM((C,), jnp.int32),
            pltpu.VMEM((C,), jnp.float32),
            pltpu.VMEM((G * W,), jnp.float32),
            pltpu.VMEM((G * W,), jnp.float32),
            pltpu.VMEM((G * C,), jnp.float32),
            pltpu.VMEM((G * C,), jnp.float32),
            pltpu.SemaphoreType.DMA,
            pltpu.SemaphoreType.DMA,
            pltpu.SemaphoreType.DMA,
            pltpu.SemaphoreType.DMA,
            pltpu.SemaphoreType.DMA,
        ],
        compiler_params=pltpu.CompilerParams(needs_layout_passes=False),
    )
    return run(input, weight_values, parent_idx.astype(jnp.int32))
